# baseline (speedup 1.0000x reference)
import jax
import jax.numpy as jnp
from jax.experimental import pallas as pl
from jax.experimental.pallas import tpu as pltpu

_K = 16
_DIL = 2
_KTOT = _K * _DIL
_NBUK = 256
_DEPTH = 16


def _knn_kernel(xq_ref, xkT_ref, out_ref):
    xq = xq_ref[0]
    xkT = xkT_ref[0]
    r = xq.shape[0]
    inner = jax.lax.dot_general(
        xq, xkT, (((1,), (0,)), ((), ())),
        preferred_element_type=jnp.float32)
    sq_q = jnp.sum(xq * xq, axis=1, keepdims=True)
    sq_k = jnp.sum(xkT * xkT, axis=0, keepdims=True)
    d = (sq_q + (-2.0) * inner) + sq_k

    big = jnp.float32(jnp.inf)
    one = jnp.int32(1)
    d4 = jnp.reshape(d, (r, _DEPTH, 2, 128))
    bm4 = jnp.min(d4, axis=1)
    lane = jax.lax.broadcasted_iota(jnp.int32, (r, 128), 1)

    a0, a1 = bm4[:, 0, :], bm4[:, 1, :]
    sel_l, sel_c = [], []
    for _ in range(_KTOT):
        lt = a1 < a0
        m = jnp.where(lt, a1, a0)
        hof = jnp.where(lt, one, 0)
        c = jnp.argmin(m, axis=1).astype(jnp.int32)[:, None]
        h = jnp.take_along_axis(hof, c, axis=1)
        sel_c.append(c)
        sel_l.append((h << 7) + c)
        eqc = lane == c
        a0 = jnp.where(eqc & (h == 0), big, a0)
        a1 = jnp.where(eqc & (h == 1), big, a1)
    s_c = jnp.concatenate(sel_c, axis=1)
    s_l = jnp.concatenate(sel_l, axis=1)

    s_hi = jnp.concatenate(sel_l, axis=1) >= 128

    parts = []
    for k in range(_DEPTH):
        glo = jnp.take_along_axis(d4[:, k, 0, :], s_c, axis=1)
        ghi = jnp.take_along_axis(d4[:, k, 1, :], s_c, axis=1)
        parts.append(jnp.where(s_hi, ghi, glo))
    pool = jnp.concatenate(parts, axis=1)
    p4 = jnp.reshape(pool, (r, 4, 128))
    p0, p1, p2, p3 = p4[:, 0, :], p4[:, 1, :], p4[:, 2, :], p4[:, 3, :]

    cols = []
    for t in range(_KTOT):
        lt01 = p1 < p0
        m01 = jnp.where(lt01, p1, p0)
        g01 = jnp.where(lt01, one, 0)
        lt23 = p3 < p2
        m23 = jnp.where(lt23, p3, p2)
        g23 = jnp.where(lt23, jnp.int32(3), jnp.int32(2))
        ltf = m23 < m01
        m = jnp.where(ltf, m23, m01)
        gof = jnp.where(ltf, g23, g01)
        c = jnp.argmin(m, axis=1).astype(jnp.int32)[:, None]
        g = jnp.take_along_axis(gof, c, axis=1)
        if t % _DIL == 0:
            p = (g << 7) + c
            s_lane = jnp.take_along_axis(s_l, p & 31, axis=1)
            cols.append(((p >> 5) << 8) + s_lane)
        eqc = lane == c
        p0 = jnp.where(eqc & (g == 0), big, p0)
        p1 = jnp.where(eqc & (g == 1), big, p1)
        p2 = jnp.where(eqc & (g == 2), big, p2)
        p3 = jnp.where(eqc & (g == 3), big, p3)
    out_ref[0] = jnp.concatenate(cols, axis=1)


def kernel(x):
    b, c, n, _ = x.shape
    xkT = x[..., 0]
    xq = jnp.swapaxes(xkT, 1, 2)

    r = 512
    grid = (b, n // r)
    nn_idx = pl.pallas_call(
        _knn_kernel,
        grid=grid,
        in_specs=[
            pl.BlockSpec((1, r, c), lambda i, j: (i, j, 0)),
            pl.BlockSpec((1, c, n), lambda i, j: (i, 0, 0)),
        ],
        out_specs=pl.BlockSpec((1, r, _K), lambda i, j: (i, j, 0)),
        out_shape=jax.ShapeDtypeStruct((b, n, _K), jnp.int32),
    )(xq, xkT)

    center_idx = jnp.broadcast_to(
        jnp.arange(n, dtype=jnp.int32)[None, :, None], (b, n, _K))
    return jnp.stack((nn_idx, center_idx), axis=0)


# speedup vs baseline: 1.1248x; 1.1248x over previous
import jax
import jax.numpy as jnp
from jax.experimental import pallas as pl
from jax.experimental.pallas import tpu as pltpu

_K = 16
_DIL = 2
_KTOT = _K * _DIL
_NBUK = 256
_DEPTH = 16


def _knn_kernel(xq_ref, xkT_ref, out_ref):
    xq = xq_ref[0]
    xkT = xkT_ref[0]
    r = xq.shape[0]
    sq_q = jnp.sum(xq * xq, axis=1, keepdims=True)
    sq_k = jnp.sum(xkT * xkT, axis=0, keepdims=True)

    big = jnp.float32(jnp.inf)
    one = jnp.int32(1)

    dks = []
    for k in range(_DEPTH):
        inner_k = jax.lax.dot_general(
            xq, xkT[:, k * _NBUK:(k + 1) * _NBUK],
            (((1,), (0,)), ((), ())),
            preferred_element_type=jnp.float32)
        dks.append((sq_q + (-2.0) * inner_k)
                   + sq_k[:, k * _NBUK:(k + 1) * _NBUK])

    bm = dks[0]
    for k in range(1, _DEPTH):
        bm = jnp.minimum(bm, dks[k])

    lane = jax.lax.broadcasted_iota(jnp.int32, (r, 128), 1)

    a0, a1 = bm[:, :128], bm[:, 128:]
    sel_c, sel_l = [], []
    for _ in range(_KTOT):
        lt = a1 < a0
        m = jnp.where(lt, a1, a0)
        hof = jnp.where(lt, one, 0)
        c = jnp.argmin(m, axis=1).astype(jnp.int32)[:, None]
        h = jnp.take_along_axis(hof, c, axis=1)
        sel_c.append(c)
        sel_l.append((h << 7) + c)
        eqc = lane == c
        a0 = jnp.where(eqc & (h == 0), big, a0)
        a1 = jnp.where(eqc & (h == 1), big, a1)
    s_c = jnp.concatenate(sel_c, axis=1)
    s_l = jnp.concatenate(sel_l, axis=1)
    s_hi = s_l >= 128

    parts = []
    for k in range(_DEPTH):
        glo = jnp.take_along_axis(dks[k][:, :128], s_c, axis=1)
        ghi = jnp.take_along_axis(dks[k][:, 128:], s_c, axis=1)
        parts.append(jnp.where(s_hi, ghi, glo))
    pool = jnp.concatenate(parts, axis=1)
    p0, p1, p2, p3 = (pool[:, 0:128], pool[:, 128:256],
                      pool[:, 256:384], pool[:, 384:512])

    cols = []
    for t in range(_KTOT):
        lt01 = p1 < p0
        m01 = jnp.where(lt01, p1, p0)
        g01 = jnp.where(lt01, one, 0)
        lt23 = p3 < p2
        m23 = jnp.where(lt23, p3, p2)
        g23 = jnp.where(lt23, jnp.int32(3), jnp.int32(2))
        ltf = m23 < m01
        m = jnp.where(ltf, m23, m01)
        gof = jnp.where(ltf, g23, g01)
        c = jnp.argmin(m, axis=1).astype(jnp.int32)[:, None]
        g = jnp.take_along_axis(gof, c, axis=1)
        if t % _DIL == 0:
            p = (g << 7) + c
            s_lane = jnp.take_along_axis(s_l, p & 31, axis=1)
            cols.append(((p >> 5) << 8) + s_lane)
        eqc = lane == c
        p0 = jnp.where(eqc & (g == 0), big, p0)
        p1 = jnp.where(eqc & (g == 1), big, p1)
        p2 = jnp.where(eqc & (g == 2), big, p2)
        p3 = jnp.where(eqc & (g == 3), big, p3)
    out_ref[0] = jnp.concatenate(cols, axis=1)


def kernel(x):
    b, c, n, _ = x.shape
    xkT = x[..., 0]
    xq = jnp.swapaxes(xkT, 1, 2)

    r = 512
    grid = (b, n // r)
    nn_idx = pl.pallas_call(
        _knn_kernel,
        grid=grid,
        in_specs=[
            pl.BlockSpec((1, r, c), lambda i, j: (i, j, 0)),
            pl.BlockSpec((1, c, n), lambda i, j: (i, 0, 0)),
        ],
        out_specs=pl.BlockSpec((1, r, _K), lambda i, j: (i, j, 0)),
        out_shape=jax.ShapeDtypeStruct((b, n, _K), jnp.int32),
    )(xq, xkT)

    center_idx = jnp.broadcast_to(
        jnp.arange(n, dtype=jnp.int32)[None, :, None], (b, n, _K))
    return jnp.stack((nn_idx, center_idx), axis=0)


# speedup vs baseline: 2.2895x; 2.0355x over previous
import jax
import jax.numpy as jnp
from jax.experimental import pallas as pl
from jax.experimental.pallas import tpu as pltpu

_K = 16
_DIL = 2
_KTOT = _K * _DIL
_NBUK = 128
_DEPTH = 32


def _knn_kernel(xq_ref, xkT_ref, out_ref):
    xq = xq_ref[0]
    xkT = xkT_ref[0]
    r = xq.shape[0]
    inner = jax.lax.dot_general(
        xq, xkT, (((1,), (0,)), ((), ())),
        preferred_element_type=jnp.float32)
    sq_q = jnp.sum(xq * xq, axis=1, keepdims=True)
    sq_k = jnp.sum(xkT * xkT, axis=0, keepdims=True)
    d = (sq_q + (-2.0) * inner) + sq_k

    big = jnp.float32(jnp.inf)
    d3 = jnp.reshape(d, (r, _DEPTH, _NBUK))
    bm = jnp.min(d3, axis=1)

    lane = jax.lax.broadcasted_iota(jnp.int32, (r, _NBUK), 1)
    sels = []
    for _ in range(_KTOT):
        c = jnp.argmin(bm, axis=1).astype(jnp.int32)[:, None]
        sels.append(c)
        bm = jnp.where(lane == c, big, bm)
    sel_lanes = jnp.concatenate(sels, axis=1)

    parts = [
        jnp.take_along_axis(d3[:, k, :], sel_lanes, axis=1)
        for k in range(_DEPTH)
    ]
    pool = jnp.concatenate(parts, axis=1)

    pos_iota = jax.lax.broadcasted_iota(jnp.int32, pool.shape, 1)
    cols = []
    for t in range(_KTOT):
        p = jnp.argmin(pool, axis=1).astype(jnp.int32)[:, None]
        if t % _DIL == 0:
            cols.append(p)
        pool = jnp.where(pos_iota == p, big, pool)
    ps = jnp.concatenate(cols, axis=1)
    s_lane = jnp.take_along_axis(sel_lanes, ps & 31, axis=1)
    out_ref[0] = ((ps >> 5) << 7) + s_lane


def kernel(x):
    b, c, n, _ = x.shape
    xkT = x[..., 0]
    xq = jnp.swapaxes(xkT, 1, 2)

    r = 1024
    grid = (b, n // r)
    nn_idx = pl.pallas_call(
        _knn_kernel,
        grid=grid,
        in_specs=[
            pl.BlockSpec((1, r, c), lambda i, j: (i, j, 0)),
            pl.BlockSpec((1, c, n), lambda i, j: (i, 0, 0)),
        ],
        out_specs=pl.BlockSpec((1, r, _K), lambda i, j: (i, j, 0)),
        out_shape=jax.ShapeDtypeStruct((b, n, _K), jnp.int32),
    )(xq, xkT)

    center_idx = jnp.broadcast_to(
        jnp.arange(n, dtype=jnp.int32)[None, :, None], (b, n, _K))
    return jnp.stack((nn_idx, center_idx), axis=0)
